# baseline (speedup 1.0000x reference)
import functools
import math

import jax
import jax.numpy as jnp
from jax import lax
from jax.experimental import pallas as pl
from jax.experimental.pallas import tpu as pltpu
from jax.experimental.pallas import tpu_sc as plsc

L = 16
NC = 2
NS = 16
NW = NC * NS


def _sc_gather_scale(ids, table, scale, B4, S, D):
    B = B4 * S
    bpw = B // NW
    wpb = S // bpw
    C = 16
    nchunk = bpw // C
    NBUF = 4
    LEAD = 2

    mesh = plsc.VectorSubcoreMesh(core_axis_name="c", subcore_axis_name="s")

    @functools.partial(
        pl.kernel,
        mesh=mesh,
        out_type=jax.ShapeDtypeStruct((B4, S, D), jnp.float32),
        scratch_types=[
            pltpu.VMEM((bpw,), jnp.int32),
            pltpu.VMEM((C, D), jnp.float32),
            pltpu.VMEM((C, D), jnp.float32),
            pltpu.VMEM((C, D), jnp.float32),
            pltpu.VMEM((C, D), jnp.float32),
            pltpu.SemaphoreType.DMA,
            pltpu.SemaphoreType.DMA,
            pltpu.SemaphoreType.DMA,
            pltpu.SemaphoreType.DMA,
            pltpu.SemaphoreType.DMA,
            pltpu.SemaphoreType.DMA,
            pltpu.SemaphoreType.DMA,
            pltpu.SemaphoreType.DMA,
        ],
    )
    def k(ids_hbm, table_hbm, out_hbm,
          idx_v, rows0, rows1, rows2, rows3,
          sg0, sg1, sg2, sg3, ss0, ss1, ss2, ss3):
        wid = lax.axis_index("s") * NC + lax.axis_index("c")
        bi = wid // wpb
        colbase = (wid % wpb) * bpw
        bufs = (rows0, rows1, rows2, rows3)
        gsems = (sg0, sg1, sg2, sg3)
        ssems = (ss0, ss1, ss2, ss3)
        pltpu.sync_copy(ids_hbm.at[bi, pl.ds(colbase, bpw)], idx_v)
        sv = jnp.full((L,), scale, jnp.float32)

        def gather(b, ci):
            return pltpu.make_async_copy(
                table_hbm.at[idx_v.at[pl.ds(ci * C, C)]], bufs[b], gsems[b])

        def store(b, ci):
            return pltpu.make_async_copy(
                bufs[b],
                out_hbm.at[bi, pl.ds(colbase + ci * C, C)],
                ssems[b])

        def half_store(b, ci, h):
            return pltpu.make_async_copy(
                bufs[b].at[pl.ds(h * (C // 2), C // 2)],
                out_hbm.at[bi, pl.ds(colbase + ci * C + h * (C // 2), C // 2)],
                ssems[b])

        for b in range(LEAD):
            gather(b, b).start()

        def step(g, carry):
            for b in range(NBUF):
                ci = g * NBUF + b
                gather(b, ci).wait()

                b2 = (b + LEAD) % NBUF

                @pl.when(ci + LEAD < nchunk)
                def _():
                    @pl.when(ci + LEAD >= NBUF)
                    def _():
                        store(b2, 0).wait()

                    gather(b2, ci + LEAD).start()

                def row(r, c2):
                    def blk(j, c3):
                        for t in range(16):
                            sl = (r, pl.ds(j * 16 * L + t * L, L))
                            bufs[b][sl] = bufs[b][sl] * sv
                        return c3
                    lax.fori_loop(0, D // L // 16, blk, 0)
                    return c2

                lax.fori_loop(0, C // 2, row, 0)
                half_store(b, ci, 0).start()
                lax.fori_loop(C // 2, C, row, 0)
                half_store(b, ci, 1).start()
            return carry

        lax.fori_loop(0, nchunk // NBUF, step, 0)
        for b in range(NBUF):
            store(b, 0).wait()

    return k(ids, table)


def kernel(freqs_cis, input_positions, embedding_table, input_token_ids, hidden_size):
    B4, S = input_token_ids.shape
    V, D = embedding_table.shape
    ids = input_token_ids
    if ids.dtype != jnp.int32:
        ids = ids.astype(jnp.int32)
    if isinstance(hidden_size, (int, float)):
        scale = math.sqrt(hidden_size)
    else:
        scale = math.sqrt(D)
    hidden_states = _sc_gather_scale(ids, embedding_table, scale, B4, S, D)
    return (freqs_cis, input_positions, hidden_states)


# speedup vs baseline: 1.1238x; 1.1238x over previous
import functools
import math

import jax
import jax.numpy as jnp
from jax import lax
from jax.experimental import pallas as pl
from jax.experimental.pallas import tpu as pltpu
from jax.experimental.pallas import tpu_sc as plsc

L = 16
NC = 2
NS = 16
NW = NC * NS


def _sc_gather_scale(ids, table, scale, B4, S, D):
    B = B4 * S
    bpw = B // NW
    wpb = S // bpw
    C = 16
    nchunk = bpw // C
    NBUF = 4
    LEAD = 2

    mesh = plsc.VectorSubcoreMesh(core_axis_name="c", subcore_axis_name="s")

    @functools.partial(
        pl.kernel,
        mesh=mesh,
        out_type=jax.ShapeDtypeStruct((B4, S, D), jnp.float32),
        scratch_types=[
            pltpu.VMEM((bpw,), jnp.int32),
            pltpu.VMEM((C, D), jnp.float32),
            pltpu.VMEM((C, D), jnp.float32),
            pltpu.VMEM((C, D), jnp.float32),
            pltpu.VMEM((C, D), jnp.float32),
            pltpu.SemaphoreType.DMA,
            pltpu.SemaphoreType.DMA,
            pltpu.SemaphoreType.DMA,
            pltpu.SemaphoreType.DMA,
            pltpu.SemaphoreType.DMA,
            pltpu.SemaphoreType.DMA,
            pltpu.SemaphoreType.DMA,
            pltpu.SemaphoreType.DMA,
        ],
    )
    def k(ids_hbm, table_hbm, out_hbm,
          idx_v, rows0, rows1, rows2, rows3,
          sg0, sg1, sg2, sg3, ss0, ss1, ss2, ss3):
        wid = lax.axis_index("s") * NC + lax.axis_index("c")
        bi = wid // wpb
        colbase = (wid % wpb) * bpw
        bufs = (rows0, rows1, rows2, rows3)
        gsems = (sg0, sg1, sg2, sg3)
        ssems = (ss0, ss1, ss2, ss3)
        pltpu.sync_copy(ids_hbm.at[bi, pl.ds(colbase, bpw)], idx_v)
        sv = jnp.full((L,), scale, jnp.float32)

        def gather(b, ci):
            return pltpu.make_async_copy(
                table_hbm.at[idx_v.at[pl.ds(ci * C, C)]], bufs[b], gsems[b])

        def store(b, ci):
            return pltpu.make_async_copy(
                bufs[b],
                out_hbm.at[bi, pl.ds(colbase + ci * C, C)],
                ssems[b])

        def half_store(b, ci, h):
            return pltpu.make_async_copy(
                bufs[b].at[pl.ds(h * (C // 2), C // 2)],
                out_hbm.at[bi, pl.ds(colbase + ci * C + h * (C // 2), C // 2)],
                ssems[b])

        for b in range(LEAD):
            gather(b, b).start()

        def step(g, carry):
            for b in range(NBUF):
                ci = g * NBUF + b
                gather(b, ci).wait()

                b2 = (b + LEAD) % NBUF

                @pl.when(ci + LEAD < nchunk)
                def _():
                    @pl.when(ci + LEAD >= NBUF)
                    def _():
                        store(b2, 0).wait()

                    gather(b2, ci + LEAD).start()

                def rowpair(p, c2):
                    r = p * 2
                    for rr in range(2):
                        for j in range(D // L):
                            sl = (r + rr, pl.ds(j * L, L))
                            bufs[b][sl] = bufs[b][sl] * sv
                    return c2

                lax.fori_loop(0, C // 4, rowpair, 0)
                half_store(b, ci, 0).start()
                lax.fori_loop(C // 4, C // 2, rowpair, 0)
                half_store(b, ci, 1).start()
            return carry

        lax.fori_loop(0, nchunk // NBUF, step, 0)
        for b in range(NBUF):
            store(b, 0).wait()

    return k(ids, table)


def kernel(freqs_cis, input_positions, embedding_table, input_token_ids, hidden_size):
    B4, S = input_token_ids.shape
    V, D = embedding_table.shape
    ids = input_token_ids
    if ids.dtype != jnp.int32:
        ids = ids.astype(jnp.int32)
    if isinstance(hidden_size, (int, float)):
        scale = math.sqrt(hidden_size)
    else:
        scale = math.sqrt(D)
    hidden_states = _sc_gather_scale(ids, embedding_table, scale, B4, S, D)
    return (freqs_cis, input_positions, hidden_states)


# speedup vs baseline: 1.1656x; 1.0372x over previous
import functools
import math

import jax
import jax.numpy as jnp
from jax import lax
from jax.experimental import pallas as pl
from jax.experimental.pallas import tpu as pltpu
from jax.experimental.pallas import tpu_sc as plsc

L = 16
NC = 2
NS = 16
NW = NC * NS


def _sc_gather_scale(ids, table, scale, B4, S, D):
    B = B4 * S
    bpw = B // NW
    wpb = S // bpw
    C = 16
    nchunk = bpw // C
    NBUF = 4
    LEAD = 2

    mesh = plsc.VectorSubcoreMesh(core_axis_name="c", subcore_axis_name="s")

    @functools.partial(
        pl.kernel,
        mesh=mesh,
        out_type=jax.ShapeDtypeStruct((B4, S, D), jnp.float32),
        scratch_types=[
            pltpu.VMEM((bpw,), jnp.int32),
            pltpu.VMEM((C, D), jnp.float32),
            pltpu.VMEM((C, D), jnp.float32),
            pltpu.VMEM((C, D), jnp.float32),
            pltpu.VMEM((C, D), jnp.float32),
            pltpu.SemaphoreType.DMA,
            pltpu.SemaphoreType.DMA,
            pltpu.SemaphoreType.DMA,
            pltpu.SemaphoreType.DMA,
            pltpu.SemaphoreType.DMA,
            pltpu.SemaphoreType.DMA,
            pltpu.SemaphoreType.DMA,
            pltpu.SemaphoreType.DMA,
        ],
    )
    def k(ids_hbm, table_hbm, out_hbm,
          idx_v, rows0, rows1, rows2, rows3,
          sg0, sg1, sg2, sg3, ss0, ss1, ss2, ss3):
        wid = lax.axis_index("s") * NC + lax.axis_index("c")
        bi = wid // wpb
        colbase = (wid % wpb) * bpw
        bufs = (rows0, rows1, rows2, rows3)
        gsems = (sg0, sg1, sg2, sg3)
        ssems = (ss0, ss1, ss2, ss3)
        pltpu.sync_copy(ids_hbm.at[bi, pl.ds(colbase, bpw)], idx_v)
        sv = jnp.full((L,), scale, jnp.float32)

        def gather(b, ci):
            return pltpu.make_async_copy(
                table_hbm.at[idx_v.at[pl.ds(ci * C, C)]], bufs[b], gsems[b])

        def store(b, ci):
            return pltpu.make_async_copy(
                bufs[b],
                out_hbm.at[bi, pl.ds(colbase + ci * C, C)],
                ssems[b])

        def half_store(b, ci, h):
            return pltpu.make_async_copy(
                bufs[b].at[pl.ds(h * (C // 2), C // 2)],
                out_hbm.at[bi, pl.ds(colbase + ci * C + h * (C // 2), C // 2)],
                ssems[b])

        for b in range(LEAD):
            gather(b, b).start()

        def step(g, carry):
            for b in range(NBUF):
                ci = g * NBUF + b
                gather(b, ci).wait()

                b2 = (b + LEAD) % NBUF

                @pl.when(ci + LEAD < nchunk)
                def _():
                    @pl.when(ci + LEAD >= NBUF)
                    def _():
                        store(b2, 0).wait()

                    gather(b2, ci + LEAD).start()

                @plsc.parallel_loop(0, C // 2)
                def _(r):
                    for j in range(D // L):
                        sl = (r, pl.ds(j * L, L))
                        bufs[b][sl] = bufs[b][sl] * sv

                half_store(b, ci, 0).start()

                @plsc.parallel_loop(C // 2, C)
                def _(r):
                    for j in range(D // L):
                        sl = (r, pl.ds(j * L, L))
                        bufs[b][sl] = bufs[b][sl] * sv

                half_store(b, ci, 1).start()
            return carry

        lax.fori_loop(0, nchunk // NBUF, step, 0)
        for b in range(NBUF):
            store(b, 0).wait()

    return k(ids, table)


def kernel(freqs_cis, input_positions, embedding_table, input_token_ids, hidden_size):
    B4, S = input_token_ids.shape
    V, D = embedding_table.shape
    ids = input_token_ids
    if ids.dtype != jnp.int32:
        ids = ids.astype(jnp.int32)
    if isinstance(hidden_size, (int, float)):
        scale = math.sqrt(hidden_size)
    else:
        scale = math.sqrt(D)
    hidden_states = _sc_gather_scale(ids, embedding_table, scale, B4, S, D)
    return (freqs_cis, input_positions, hidden_states)


# speedup vs baseline: 1.1815x; 1.0136x over previous
import functools
import math

import jax
import jax.numpy as jnp
from jax import lax
from jax.experimental import pallas as pl
from jax.experimental.pallas import tpu as pltpu
from jax.experimental.pallas import tpu_sc as plsc

L = 16
NC = 2
NS = 16
NW = NC * NS


def _sc_gather_scale(ids, table, scale, B4, S, D):
    B = B4 * S
    bpw = B // NW
    wpb = S // bpw
    C = 16
    nchunk = bpw // C
    NBUF = 4
    LEAD = 2

    mesh = plsc.VectorSubcoreMesh(core_axis_name="c", subcore_axis_name="s")

    @functools.partial(
        pl.kernel,
        mesh=mesh,
        out_type=jax.ShapeDtypeStruct((B4, S, D), jnp.float32),
        scratch_types=[
            pltpu.VMEM((bpw,), jnp.int32),
            pltpu.VMEM((C, D), jnp.float32),
            pltpu.VMEM((C, D), jnp.float32),
            pltpu.VMEM((C, D), jnp.float32),
            pltpu.VMEM((C, D), jnp.float32),
            pltpu.SemaphoreType.DMA,
            pltpu.SemaphoreType.DMA,
            pltpu.SemaphoreType.DMA,
            pltpu.SemaphoreType.DMA,
            pltpu.SemaphoreType.DMA,
            pltpu.SemaphoreType.DMA,
            pltpu.SemaphoreType.DMA,
            pltpu.SemaphoreType.DMA,
        ],
    )
    def k(ids_hbm, table_hbm, out_hbm,
          idx_v, rows0, rows1, rows2, rows3,
          sg0, sg1, sg2, sg3, ss0, ss1, ss2, ss3):
        wid = lax.axis_index("s") * NC + lax.axis_index("c")
        bi = wid // wpb
        colbase = (wid % wpb) * bpw
        bufs = (rows0, rows1, rows2, rows3)
        gsems = (sg0, sg1, sg2, sg3)
        ssems = (ss0, ss1, ss2, ss3)
        pltpu.sync_copy(ids_hbm.at[bi, pl.ds(colbase, bpw)], idx_v)
        sv = jnp.full((L,), scale, jnp.float32)

        def gather(b, ci):
            return pltpu.make_async_copy(
                table_hbm.at[idx_v.at[pl.ds(ci * C, C)]], bufs[b], gsems[b])

        def store(b, ci):
            return pltpu.make_async_copy(
                bufs[b],
                out_hbm.at[bi, pl.ds(colbase + ci * C, C)],
                ssems[b])

        def half_store(b, ci, h):
            return pltpu.make_async_copy(
                bufs[b].at[pl.ds(h * (C // 2), C // 2)],
                out_hbm.at[bi, pl.ds(colbase + ci * C + h * (C // 2), C // 2)],
                ssems[b])

        for b in range(LEAD):
            gather(b, b).start()

        def step(g, carry):
            for b in range(NBUF):
                ci = g * NBUF + b
                gather(b, ci).wait()

                b2 = (b + LEAD) % NBUF

                @pl.when(ci + LEAD < nchunk)
                def _():
                    @pl.when(ci + LEAD >= NBUF)
                    def _():
                        store(b2, 0).wait()

                    gather(b2, ci + LEAD).start()

                def row(r, c2):
                    for j in range(D // L):
                        sl = (r, pl.ds(j * L, L))
                        bufs[b][sl] = bufs[b][sl] * sv
                    return c2

                lax.fori_loop(0, C // 2, row, 0)
                half_store(b, ci, 0).start()
                lax.fori_loop(C // 2, C, row, 0)
                half_store(b, ci, 1).start()
            return carry

        lax.fori_loop(0, nchunk // NBUF, step, 0)
        for b in range(NBUF):
            store(b, 0).wait()

    return k(ids, table)


def kernel(freqs_cis, input_positions, embedding_table, input_token_ids, hidden_size):
    B4, S = input_token_ids.shape
    V, D = embedding_table.shape
    ids = input_token_ids
    if ids.dtype != jnp.int32:
        ids = ids.astype(jnp.int32)
    if isinstance(hidden_size, (int, float)):
        scale = math.sqrt(hidden_size)
    else:
        scale = math.sqrt(D)
    hidden_states = _sc_gather_scale(ids, embedding_table, scale, B4, S, D)
    return (freqs_cis, input_positions, hidden_states)
